# baseline (speedup 1.0000x reference)
import functools

import jax
import jax.numpy as jnp
from jax import lax
from jax.experimental import pallas as pl
from jax.experimental.pallas import tpu as pltpu
from jax.experimental.pallas import tpu_sc as plsc

VOCAB = 100000
D = 128
H = 32
C = 1000
B = 4096
L = 50

NC = 2
NS = 16
LANES = 16
NW = NC * NS
SPW = B // NW
PAIRS = SPW // 2
ROWS = 2 * L
BT = 512


def _pool_body(e1_hbm, e2_hbm, e3_hbm, idx_hbm, out_hbm,
               idx_v, rows0, rows1, rows2, rows3, acc_a, acc_b,
               sem0, sem1, sem2, sem3, wsem_a, wsem_b):
    wid = lax.axis_index("s") * NC + lax.axis_index("c")
    nch = D // LANES
    bufs = ((rows0, sem0), (rows1, sem1), (rows2, sem2), (rows3, sem3))
    embs = (e1_hbm, e2_hbm, e3_hbm)
    accs = (acc_a, acc_b, acc_a)
    wsems = (wsem_a, wsem_b, wsem_a)

    for t in range(3):
        pltpu.sync_copy(idx_hbm.at[t, wid], idx_v.at[pl.ds(t * PAIRS, PAIRS)])

    for b, (buf, sem) in enumerate(bufs):
        pltpu.async_copy(embs[0].at[idx_v.at[b]], buf, sem)

    for t, emb in enumerate(embs):
        acc_v = accs[t]
        if t == 2:
            pltpu.make_async_copy(
                acc_v, out_hbm.at[0].at[pl.ds(wid * SPW, SPW)],
                wsem_a).wait()

        @pl.loop(0, PAIRS, step=4)
        def _(j, t=t, emb=emb, acc_v=acc_v):
            for b, (buf, sem) in enumerate(bufs):
                jj = j + b
                pltpu.make_async_copy(
                    emb.at[idx_v.at[0]], buf, sem).wait()
                for s in range(2):
                    def body(l, carry, s=s, buf=buf):
                        r = s * L + 2 * l
                        out = []
                        for c in range(nch):
                            sl = pl.ds(c * LANES, LANES)
                            out.append(carry[c] + buf[r, sl] + buf[r + 1, sl])
                        return tuple(out)
                    acc = lax.fori_loop(
                        0, L // 2, body,
                        tuple(jnp.zeros((LANES,), jnp.float32)
                              for _ in range(nch)))
                    for c in range(nch):
                        acc_v[2 * jj + s, pl.ds(c * LANES, LANES)] = acc[c]
                nxt = j + b + 4
                row = t * PAIRS + nxt

                @pl.when(nxt < PAIRS)
                def _(emb=emb, buf=buf, sem=sem, row=row):
                    pltpu.async_copy(emb.at[idx_v.at[row]], buf, sem)

                if t < 2:
                    nemb = embs[t + 1]

                    @pl.when(nxt >= PAIRS)
                    def _(nemb=nemb, buf=buf, sem=sem, row=row):
                        pltpu.async_copy(nemb.at[idx_v.at[row]], buf, sem)

        pltpu.async_copy(
            acc_v, out_hbm.at[t].at[pl.ds(wid * SPW, SPW)], wsems[t])
        if t == 2:
            pltpu.make_async_copy(
                acc_b, out_hbm.at[1].at[pl.ds(wid * SPW, SPW)],
                wsem_b).wait()
            pltpu.make_async_copy(
                acc_a, out_hbm.at[2].at[pl.ds(wid * SPW, SPW)],
                wsem_a).wait()


def _mlp_body(p_ref, w1_ref, b1_ref, w2_ref, b2_ref, o_ref):
    p = p_ref[...].astype(jnp.bfloat16)
    w1 = w1_ref[...].astype(jnp.bfloat16)
    h = (jnp.dot(p[0], w1[0:D], preferred_element_type=jnp.float32)
         + jnp.dot(p[1], w1[D:2 * D], preferred_element_type=jnp.float32)
         + jnp.dot(p[2], w1[2 * D:], preferred_element_type=jnp.float32))
    h = jnp.maximum(h * (1.0 / L) + b1_ref[...], 0.0)
    o_ref[...] = (jnp.dot(h.astype(jnp.bfloat16),
                          w2_ref[...].astype(jnp.bfloat16),
                          preferred_element_type=jnp.float32)
                  + b2_ref[...])


def kernel(x, emb1, emb2, emb3, W1, b1, W2, b2):
    x = x.astype(jnp.int32)
    idx = x.reshape(3, NW, PAIRS, ROWS)

    mesh = plsc.VectorSubcoreMesh(core_axis_name="c", subcore_axis_name="s")
    pooled = pl.kernel(
        _pool_body,
        out_type=jax.ShapeDtypeStruct((3, B, D), jnp.float32),
        mesh=mesh,
        scratch_types=[
            pltpu.VMEM((3 * PAIRS, ROWS), jnp.int32),
            pltpu.VMEM((ROWS, D), jnp.float32),
            pltpu.VMEM((ROWS, D), jnp.float32),
            pltpu.VMEM((ROWS, D), jnp.float32),
            pltpu.VMEM((ROWS, D), jnp.float32),
            pltpu.VMEM((SPW, D), jnp.float32),
            pltpu.VMEM((SPW, D), jnp.float32),
            pltpu.SemaphoreType.DMA,
            pltpu.SemaphoreType.DMA,
            pltpu.SemaphoreType.DMA,
            pltpu.SemaphoreType.DMA,
            pltpu.SemaphoreType.DMA,
            pltpu.SemaphoreType.DMA,
        ],
    )(emb1, emb2, emb3, idx)

    b1r = b1.reshape(1, H)
    b2r = b2.reshape(1, C)

    out = pl.pallas_call(
        _mlp_body,
        grid=(B // BT,),
        in_specs=[
            pl.BlockSpec((3, BT, D), lambda i: (0, i, 0)),
            pl.BlockSpec((3 * D, H), lambda i: (0, 0)),
            pl.BlockSpec((1, H), lambda i: (0, 0)),
            pl.BlockSpec((H, C), lambda i: (0, 0)),
            pl.BlockSpec((1, C), lambda i: (0, 0)),
        ],
        out_specs=pl.BlockSpec((BT, C), lambda i: (i, 0)),
        out_shape=jax.ShapeDtypeStruct((B, C), jnp.float32),
    )(pooled, W1, b1r, W2, b2r)
    return out


# speedup vs baseline: 1.0053x; 1.0053x over previous
import jax
import jax.numpy as jnp
from jax import lax
from jax.experimental import pallas as pl
from jax.experimental.pallas import tpu as pltpu
from jax.experimental.pallas import tpu_sc as plsc

VOCAB = 100000
D = 128
H = 32
C = 1000
B = 4096
L = 50

NC = 2
NS = 16
LANES = 16
NW = NC * NS
SPW = B // NW
PAIRS = SPW // 2
ROWS = 2 * L
BT = 512


def _pool_body(e1_hbm, e2_hbm, e3_hbm, idx_hbm, out_hbm,
               idx_v, rows0, rows1, rows2, rows3, acc_a, acc_b,
               sem0, sem1, sem2, sem3, wsem_a, wsem_b):
    wid = lax.axis_index("s") * NC + lax.axis_index("c")
    nch = D // LANES
    bufs = ((rows0, sem0), (rows1, sem1), (rows2, sem2), (rows3, sem3))
    embs = (e1_hbm, e2_hbm, e3_hbm)
    accs = (acc_a, acc_b, acc_a)
    wsems = (wsem_a, wsem_b, wsem_a)

    for t in range(3):
        pltpu.sync_copy(idx_hbm.at[t, wid], idx_v.at[pl.ds(t * PAIRS, PAIRS)])

    for b, (buf, sem) in enumerate(bufs):
        pltpu.async_copy(embs[0].at[idx_v.at[b]], buf, sem)

    for t, emb in enumerate(embs):
        acc_v = accs[t]
        if t == 2:
            pltpu.make_async_copy(
                acc_v, out_hbm.at[0].at[pl.ds(wid * SPW, SPW)],
                wsem_a).wait()

        @pl.loop(0, PAIRS, step=4)
        def _(j, t=t, emb=emb, acc_v=acc_v):
            for b, (buf, sem) in enumerate(bufs):
                jj = j + b
                pltpu.make_async_copy(
                    emb.at[idx_v.at[0]], buf, sem).wait()
                for s in range(2):
                    def body(l, carry, s=s, buf=buf):
                        r = s * L + 2 * l
                        out = []
                        for c in range(nch):
                            sl = pl.ds(c * LANES, LANES)
                            out.append(carry[c] + buf[r, sl] + buf[r + 1, sl])
                        return tuple(out)
                    acc = lax.fori_loop(
                        0, L // 2, body,
                        tuple(jnp.zeros((LANES,), jnp.float32)
                              for _ in range(nch)))
                    for c in range(nch):
                        acc_v[2 * jj + s, pl.ds(c * LANES, LANES)] = acc[c]
                nxt = j + b + 4
                row = t * PAIRS + nxt

                @pl.when(nxt < PAIRS)
                def _(emb=emb, buf=buf, sem=sem, row=row):
                    pltpu.async_copy(emb.at[idx_v.at[row]], buf, sem)

                if t < 2:
                    nemb = embs[t + 1]

                    @pl.when(nxt >= PAIRS)
                    def _(nemb=nemb, buf=buf, sem=sem, row=row):
                        pltpu.async_copy(nemb.at[idx_v.at[row]], buf, sem)

        pltpu.async_copy(
            acc_v, out_hbm.at[t].at[pl.ds(wid * SPW, SPW)], wsems[t])
        if t == 2:
            pltpu.make_async_copy(
                acc_b, out_hbm.at[1].at[pl.ds(wid * SPW, SPW)],
                wsem_b).wait()
            pltpu.make_async_copy(
                acc_a, out_hbm.at[2].at[pl.ds(wid * SPW, SPW)],
                wsem_a).wait()


def _mlp_body(p_ref, w1_ref, b1_ref, w2_ref, b2_ref, o_ref):
    p = p_ref[...]
    w1 = w1_ref[...]
    h = (jnp.dot(p[0], w1[0:D], preferred_element_type=jnp.float32)
         + jnp.dot(p[1], w1[D:2 * D], preferred_element_type=jnp.float32)
         + jnp.dot(p[2], w1[2 * D:], preferred_element_type=jnp.float32))
    h = jnp.maximum(h * (1.0 / L) + b1_ref[...], 0.0)
    o_ref[...] = (jnp.dot(h, w2_ref[...], preferred_element_type=jnp.float32)
                  + b2_ref[...])


def kernel(x, emb1, emb2, emb3, W1, b1, W2, b2):
    x = x.astype(jnp.int32)
    idx = x.reshape(3, NW, PAIRS, ROWS)

    mesh = plsc.VectorSubcoreMesh(core_axis_name="c", subcore_axis_name="s")
    pooled = pl.kernel(
        _pool_body,
        out_type=jax.ShapeDtypeStruct((3, B, D), jnp.float32),
        mesh=mesh,
        scratch_types=[
            pltpu.VMEM((3 * PAIRS, ROWS), jnp.int32),
            pltpu.VMEM((ROWS, D), jnp.float32),
            pltpu.VMEM((ROWS, D), jnp.float32),
            pltpu.VMEM((ROWS, D), jnp.float32),
            pltpu.VMEM((ROWS, D), jnp.float32),
            pltpu.VMEM((SPW, D), jnp.float32),
            pltpu.VMEM((SPW, D), jnp.float32),
            pltpu.SemaphoreType.DMA,
            pltpu.SemaphoreType.DMA,
            pltpu.SemaphoreType.DMA,
            pltpu.SemaphoreType.DMA,
            pltpu.SemaphoreType.DMA,
            pltpu.SemaphoreType.DMA,
        ],
    )(emb1, emb2, emb3, idx)

    b1r = b1.reshape(1, H)
    b2r = b2.reshape(1, C)

    out = pl.pallas_call(
        _mlp_body,
        grid=(B // BT,),
        in_specs=[
            pl.BlockSpec((3, BT, D), lambda i: (0, i, 0)),
            pl.BlockSpec((3 * D, H), lambda i: (0, 0)),
            pl.BlockSpec((1, H), lambda i: (0, 0)),
            pl.BlockSpec((H, C), lambda i: (0, 0)),
            pl.BlockSpec((1, C), lambda i: (0, 0)),
        ],
        out_specs=pl.BlockSpec((BT, C), lambda i: (i, 0)),
        out_shape=jax.ShapeDtypeStruct((B, C), jnp.float32),
    )(pooled, W1, b1r, W2, b2r)
    return out
